# baseline (speedup 1.0000x reference)
import jax
import jax.numpy as jnp
from jax.experimental import pallas as pl
from jax.experimental.pallas import tpu as pltpu

B, L, D = 4, 512, 140
U, T, NB = 4, 10, 2


def _gcnn_kernel(nodes_ref, adj_hbm, wio_ref, bio_ref, wg_ref, bg_ref, out_ref,
                 abuf, sem):
    bf = jnp.bfloat16
    h = nodes_ref[0]

    b = pl.program_id(0)
    slot = jax.lax.rem(b, 2)
    nslot = jax.lax.rem(b + 1, 2)

    @pl.when(b == 0)
    def _():
        pltpu.make_async_copy(adj_hbm.at[0], abuf.at[0], sem.at[0]).start()

    @pl.when(b + 1 < B)
    def _():
        pltpu.make_async_copy(adj_hbm.at[b + 1], abuf.at[nslot],
                              sem.at[nslot]).start()

    pltpu.make_async_copy(adj_hbm.at[b], abuf.at[slot], sem.at[slot]).wait()
    adjb = abuf.at[slot]

    adj_rare = adjb[U]
    for t in range(U + 1, T):
        adj_rare = adj_rare + adjb[t]

    a_bf = [adjb[t].astype(bf) for t in range(U)] + [adj_rare.astype(bf)]

    for l in range(NB):
        h_bf = h.astype(bf)
        p = jnp.dot(h_bf, wio_ref[l], preferred_element_type=jnp.float32)
        p = p + bio_ref[l]
        g = jax.nn.sigmoid(
            jnp.dot(h_bf, wg_ref[l], preferred_element_type=jnp.float32)
            + bg_ref[l])

        vin = [(p[:, t * D:(t + 1) * D] * g[:, t:t + 1]).astype(bf)
               for t in range(U)]
        vin.append((p[:, 2 * U * D:2 * U * D + D] * g[:, 2 * U:2 * U + 1])
                   .astype(bf))
        vout = [(p[:, (U + t) * D:(U + t + 1) * D] * g[:, U + t:U + t + 1])
                .astype(bf) for t in range(U)]
        vout.append((p[:, 2 * U * D + D:2 * U * D + 2 * D]
                     * g[:, 2 * U + 1:2 * U + 2]).astype(bf))

        acc_in = jnp.dot(a_bf[0], vin[0], preferred_element_type=jnp.float32)
        for t in range(1, U + 1):
            acc_in = acc_in + jnp.dot(a_bf[t], vin[t],
                                      preferred_element_type=jnp.float32)

        acc_out_t = jax.lax.dot_general(
            vout[0], a_bf[0], (((0,), (0,)), ((), ())),
            preferred_element_type=jnp.float32)
        for t in range(1, U + 1):
            acc_out_t = acc_out_t + jax.lax.dot_general(
                vout[t], a_bf[t], (((0,), (0,)), ((), ())),
                preferred_element_type=jnp.float32)

        h = jnp.maximum(acc_in + acc_out_t.T, 0.0) + h

    out_ref[0] = h


def kernel(nodes_embed, adj, Win_w, Win_b, Wout_w, Wout_b, Wing_w, Wing_b,
           Woutg_w, Woutg_b, Rin_w, Rin_b, Rout_w, Rout_b, Ring_w, Ring_b,
           Routg_w, Routg_b):
    wio = jnp.concatenate([Win_w, Wout_w, Rin_w, Rout_w], axis=2).astype(
        jnp.bfloat16)
    bio = jnp.concatenate([Win_b, Wout_b, Rin_b, Rout_b], axis=1)
    wg = jnp.concatenate([Wing_w, Woutg_w, Ring_w, Routg_w], axis=2).astype(
        jnp.bfloat16)
    bg = jnp.concatenate([Wing_b, Woutg_b, Ring_b, Routg_b], axis=1)

    return pl.pallas_call(
        _gcnn_kernel,
        grid=(B,),
        in_specs=[
            pl.BlockSpec((1, L, D), lambda b: (b, 0, 0)),
            pl.BlockSpec(memory_space=pltpu.MemorySpace.HBM),
            pl.BlockSpec((NB, D, 2 * U * D + 2 * D), lambda b: (0, 0, 0)),
            pl.BlockSpec((NB, 2 * U * D + 2 * D), lambda b: (0, 0)),
            pl.BlockSpec((NB, D, 2 * U + 2), lambda b: (0, 0, 0)),
            pl.BlockSpec((NB, 2 * U + 2), lambda b: (0, 0)),
        ],
        out_specs=pl.BlockSpec((1, L, D), lambda b: (b, 0, 0)),
        out_shape=jax.ShapeDtypeStruct((B, L, D), jnp.float32),
        scratch_shapes=[
            pltpu.VMEM((2, T, L, L), jnp.float32),
            pltpu.SemaphoreType.DMA((2,)),
        ],
        compiler_params=pltpu.CompilerParams(
            dimension_semantics=("arbitrary",),
            vmem_limit_bytes=100 * 1024 * 1024,
        ),
    )(nodes_embed, adj, wio, bio, wg, bg)


# speedup vs baseline: 1.0152x; 1.0152x over previous
import jax
import jax.numpy as jnp
from jax.experimental import pallas as pl
from jax.experimental.pallas import tpu as pltpu

B, L, D = 4, 512, 140
U, T, NB = 4, 10, 2
DP = 144
NP = 2 * U + 2
IN_PIECES = (0, 1, 2, 3, 8)
OUT_PIECES = (4, 5, 6, 7, 9)


def _gcnn_kernel(nodes_ref, adj_ref, wioT_ref, wgT_ref, out_ref,
                 aflat_ref, atflat_ref):
    bf = jnp.bfloat16
    adjb = adj_ref[0]

    rare = ((adjb[U] + adjb[U + 1]) + (adjb[U + 2] + adjb[U + 3])) \
        + (adjb[U + 4] + adjb[U + 5])
    for t in range(U):
        a_bf = adjb[t].astype(bf)
        aflat_ref[t * L:(t + 1) * L, :] = a_bf
        atflat_ref[t * L:(t + 1) * L, :] = a_bf.T
    rare_bf = rare.astype(bf)
    aflat_ref[U * L:(U + 1) * L, :] = rare_bf
    atflat_ref[U * L:(U + 1) * L, :] = rare_bf.T

    hT = nodes_ref[0].T
    ones_row = jnp.ones((1, L), dtype=bf)

    for l in range(NB):
        aug = jnp.concatenate([hT.astype(bf), ones_row], axis=0)
        pT = jnp.dot(wioT_ref[l], aug, preferred_element_type=jnp.float32)
        gT = jax.nn.sigmoid(
            jnp.dot(wgT_ref[l], aug, preferred_element_type=jnp.float32))

        gin = [(pT[DP * p:DP * p + D, :] * gT[p:p + 1, :]).astype(bf)
               for p in IN_PIECES]
        acc_inT = jnp.dot(jnp.concatenate(gin, axis=1), atflat_ref[...],
                          preferred_element_type=jnp.float32)
        gout = [(pT[DP * p:DP * p + D, :] * gT[p:p + 1, :]).astype(bf)
                for p in OUT_PIECES]
        acc_outT = jnp.dot(jnp.concatenate(gout, axis=1), aflat_ref[...],
                           preferred_element_type=jnp.float32)

        hT = jnp.maximum(acc_inT + acc_outT, 0.0) + hT

    out_ref[0] = hT.T


def kernel(nodes_embed, adj, Win_w, Win_b, Wout_w, Wout_b, Wing_w, Wing_b,
           Woutg_w, Woutg_b, Rin_w, Rin_b, Rout_w, Rout_b, Ring_w, Ring_b,
           Routg_w, Routg_b):
    w_pieces = [Win_w[:, :, t * D:(t + 1) * D] for t in range(U)] \
        + [Wout_w[:, :, t * D:(t + 1) * D] for t in range(U)] \
        + [Rin_w, Rout_w]
    b_pieces = [Win_b[:, t * D:(t + 1) * D] for t in range(U)] \
        + [Wout_b[:, t * D:(t + 1) * D] for t in range(U)] \
        + [Rin_b, Rout_b]
    blocks = []
    for wp, bp in zip(w_pieces, b_pieces):
        blk = jnp.concatenate([wp.transpose(0, 2, 1), bp[:, :, None]], axis=2)
        blocks.append(jnp.pad(blk, ((0, 0), (0, DP - D), (0, 0))))
    wioT = jnp.concatenate(blocks, axis=1).astype(jnp.bfloat16)

    wg = jnp.concatenate([Wing_w, Woutg_w, Ring_w, Routg_w], axis=2)
    bg = jnp.concatenate([Wing_b, Woutg_b, Ring_b, Routg_b], axis=1)
    wgT = jnp.concatenate([wg.transpose(0, 2, 1), bg[:, :, None]],
                          axis=2).astype(jnp.bfloat16)

    return pl.pallas_call(
        _gcnn_kernel,
        grid=(B,),
        in_specs=[
            pl.BlockSpec((1, L, D), lambda b: (b, 0, 0)),
            pl.BlockSpec((1, T, L, L), lambda b: (b, 0, 0, 0)),
            pl.BlockSpec((NB, NP * DP, D + 1), lambda b: (0, 0, 0)),
            pl.BlockSpec((NB, NP, D + 1), lambda b: (0, 0, 0)),
        ],
        out_specs=pl.BlockSpec((1, L, D), lambda b: (b, 0, 0)),
        out_shape=jax.ShapeDtypeStruct((B, L, D), jnp.float32),
        scratch_shapes=[
            pltpu.VMEM(((U + 1) * L, L), jnp.bfloat16),
            pltpu.VMEM(((U + 1) * L, L), jnp.bfloat16),
        ],
        compiler_params=pltpu.CompilerParams(
            dimension_semantics=("arbitrary",),
            vmem_limit_bytes=100 * 1024 * 1024,
        ),
    )(nodes_embed, adj, wioT, wgT)
